# Initial kernel scaffold; baseline (speedup 1.0000x reference)
import jax
import jax.numpy as jnp
from jax.experimental import pallas as pl


def kernel(z, edge_index):
    raise NotImplementedError("write your pallas kernel here")


# speedup vs baseline: 1.0976x; 1.0976x over previous
import functools

import jax
import jax.numpy as jnp
from jax import lax
from jax.experimental import pallas as pl
from jax.experimental.pallas import tpu as pltpu
from jax.experimental.pallas import tpu_sc as plsc

_L = 16


@functools.lru_cache(maxsize=None)
def _make_kernel(N, D, E):
    NC, NS = 2, 16
    NW = NC * NS
    CHUNK = 80
    EP = E // NW
    NCHUNK = EP // CHUNK
    assert EP * NW == E and NCHUNK * CHUNK == EP

    mesh = plsc.VectorSubcoreMesh(core_axis_name="c", subcore_axis_name="s")

    @functools.partial(
        pl.kernel,
        mesh=mesh,
        compiler_params=pltpu.CompilerParams(needs_layout_passes=False),
        out_type=jax.ShapeDtypeStruct((E,), jnp.float32),
        scratch_types=[
            pltpu.VMEM((CHUNK,), jnp.int32),
            pltpu.VMEM((CHUNK,), jnp.int32),
            pltpu.VMEM((CHUNK, D), jnp.float32),
            pltpu.VMEM((CHUNK, D), jnp.float32),
            pltpu.VMEM((CHUNK,), jnp.float32),
            pltpu.SemaphoreType.DMA,
            pltpu.SemaphoreType.DMA,
        ],
    )
    def k(z_hbm, src_hbm, dst_hbm, out_hbm, sidx, didx, srows, drows, obuf,
          sem_s, sem_d):
        wid = lax.axis_index("s") * NC + lax.axis_index("c")
        base = wid * EP

        def chunk_body(c, carry):
            off = base + c * CHUNK
            pltpu.sync_copy(src_hbm.at[pl.ds(off, CHUNK)], sidx)
            pltpu.sync_copy(dst_hbm.at[pl.ds(off, CHUNK)], didx)
            cp_s = pltpu.async_copy(z_hbm.at[sidx], srows, sem_s)
            cp_d = pltpu.async_copy(z_hbm.at[didx], drows, sem_d)
            cp_s.wait()
            cp_d.wait()

            lane = lax.iota(jnp.int32, _L)

            def group_body(g, c2):
                rows = g * _L + lane
                acc = jnp.zeros((_L,), jnp.float32)
                col = jnp.zeros((_L,), jnp.int32)
                for d in range(D):
                    s = plsc.load_gather(srows, [rows, col])
                    t = plsc.load_gather(drows, [rows, col])
                    acc = acc + s * t
                    col = col + 1
                obuf[pl.ds(g * _L, _L)] = 1.0 / (1.0 + jnp.exp(-acc))
                return c2

            lax.fori_loop(0, CHUNK // _L, group_body, 0)
            pltpu.sync_copy(obuf, out_hbm.at[pl.ds(off, CHUNK)])
            return carry

        lax.fori_loop(0, NCHUNK, chunk_body, 0)

    return k


def kernel(z, edge_index):
    N, D = z.shape
    E = edge_index.shape[1]
    ei = edge_index.astype(jnp.int32)
    k = _make_kernel(N, D, E)
    return k(z, ei[0], ei[1])


# speedup vs baseline: 4.0459x; 3.6862x over previous
import functools

import jax
import jax.numpy as jnp
from jax import lax
from jax.experimental import pallas as pl
from jax.experimental.pallas import tpu as pltpu
from jax.experimental.pallas import tpu_sc as plsc

_L = 16


@functools.lru_cache(maxsize=None)
def _make_kernel(N, D, E):
    NC, NS = 2, 16
    NW = NC * NS
    CHUNK = 80
    EP = E // NW
    NCHUNK = EP // CHUNK
    assert EP * NW == E and NCHUNK * CHUNK == EP and NCHUNK % 2 == 1
    NG = CHUNK // _L

    mesh = plsc.VectorSubcoreMesh(core_axis_name="c", subcore_axis_name="s")

    @functools.partial(
        pl.kernel,
        mesh=mesh,
        compiler_params=pltpu.CompilerParams(needs_layout_passes=False),
        out_type=jax.ShapeDtypeStruct((E,), jnp.float32),
        scratch_types=[
            pltpu.VMEM((EP,), jnp.int32),
            pltpu.VMEM((EP,), jnp.int32),
            pltpu.VMEM((CHUNK, D), jnp.float32),
            pltpu.VMEM((CHUNK, D), jnp.float32),
            pltpu.VMEM((CHUNK, D), jnp.float32),
            pltpu.VMEM((CHUNK, D), jnp.float32),
            pltpu.VMEM((EP,), jnp.float32),
            pltpu.SemaphoreType.DMA,
            pltpu.SemaphoreType.DMA,
        ],
    )
    def k(z_hbm, src_hbm, dst_hbm, out_hbm, sidx, didx,
          srows_a, drows_a, srows_b, drows_b, oall, sem_a, sem_b):
        wid = lax.axis_index("s") * NC + lax.axis_index("c")
        base = wid * EP
        pltpu.sync_copy(src_hbm.at[pl.ds(base, EP)], sidx)
        pltpu.sync_copy(dst_hbm.at[pl.ds(base, EP)], didx)

        lane = lax.iota(jnp.int32, _L)

        def fire(c, srows, drows, sem):
            sl = pl.ds(c * CHUNK, CHUNK)
            pltpu.async_copy(z_hbm.at[sidx.at[sl]], srows, sem)
            pltpu.async_copy(z_hbm.at[didx.at[sl]], drows, sem)

        def drain(srows, drows, sem):
            sl = pl.ds(0, CHUNK)
            pltpu.make_async_copy(z_hbm.at[sidx.at[sl]], srows, sem).wait()
            pltpu.make_async_copy(z_hbm.at[didx.at[sl]], drows, sem).wait()

        def compute(c, srows, drows):
            for g in range(NG):
                vec = jnp.zeros((_L,), jnp.float32)
                for m in range(_L):
                    e = g * _L + m
                    acc = srows[e, pl.ds(0, _L)] * drows[e, pl.ds(0, _L)]
                    for j in range(1, D // _L):
                        acc = acc + (srows[e, pl.ds(j * _L, _L)] *
                                     drows[e, pl.ds(j * _L, _L)])
                    vec = jnp.where(lane == m, jnp.sum(acc), vec)
                off = c * CHUNK + g * _L
                oall[pl.ds(off, _L)] = 1.0 / (1.0 + jnp.exp(-vec))

        fire(0, srows_a, drows_a, sem_a)

        def body(kk, carry):
            c = 2 * kk
            fire(c + 1, srows_b, drows_b, sem_b)
            drain(srows_a, drows_a, sem_a)
            compute(c, srows_a, drows_a)

            @pl.when(c + 2 < NCHUNK)
            def _():
                fire(c + 2, srows_a, drows_a, sem_a)

            drain(srows_b, drows_b, sem_b)
            compute(c + 1, srows_b, drows_b)
            return carry

        lax.fori_loop(0, (NCHUNK - 1) // 2, body, 0)
        drain(srows_a, drows_a, sem_a)
        compute(NCHUNK - 1, srows_a, drows_a)

        pltpu.sync_copy(oall, out_hbm.at[pl.ds(base, EP)])

    return k


def kernel(z, edge_index):
    N, D = z.shape
    E = edge_index.shape[1]
    ei = edge_index.astype(jnp.int32)
    k = _make_kernel(N, D, E)
    return k(z, ei[0], ei[1])
